# baseline (speedup 1.0000x reference)
import functools

import jax
import jax.numpy as jnp
from jax import lax
from jax.experimental import pallas as pl
from jax.experimental.pallas import tpu as pltpu
from jax.experimental.pallas import tpu_sc as plsc

B, S, V, D, P, T = 1024, 200, 100000, 128, 512, 2
N = B * S
NC, NS, L = 2, 16, 16
NW = NC * NS
W = N // NW
CH = 128
NCHUNK = W // CH
NJ = D // L
EPS = 1e-12

_mesh = plsc.VectorSubcoreMesh(core_axis_name="c", subcore_axis_name="s")


def _rsqrt16(x):
    i = lax.bitcast_convert_type(x, jnp.int32)
    i = jnp.int32(0x5F3759DF) - lax.shift_right_logical(i, 1)
    y = lax.bitcast_convert_type(i, jnp.float32)
    h = x * jnp.float32(-0.5)
    for _ in range(2):
        y = y * (jnp.float32(1.5) + h * y * y)
    return y


def _tree_add(vs):
    vs = list(vs)
    while len(vs) > 1:
        vs = [vs[i] + vs[i + 1] for i in range(0, len(vs) - 1, 2)] + (
            [vs[-1]] if len(vs) % 2 else [])
    return vs[0]


@functools.partial(
    pl.kernel,
    out_type=jax.ShapeDtypeStruct((N, D), jnp.float32),
    mesh=_mesh,
    scratch_types=[
        pltpu.VMEM((2, CH, D), jnp.float32),
        pltpu.VMEM((2, CH, D), jnp.float32),
        pltpu.VMEM((2, CH, D), jnp.float32),
        pltpu.VMEM((2, 3, CH), jnp.int32),
        pltpu.VMEM((2, CH), jnp.int32),
        pltpu.VMEM((2, CH), jnp.int32),
        pltpu.VMEM((D,), jnp.float32),
        pltpu.VMEM((D,), jnp.float32),
        pltpu.SemaphoreType.DMA,
        pltpu.SemaphoreType.DMA,
        pltpu.SemaphoreType.DMA,
        pltpu.SemaphoreType.DMA,
        pltpu.SemaphoreType.DMA,
        pltpu.SemaphoreType.DMA,
    ],
)
def _sc_embed(ids3, tok_tab, comb_tab, gamma, beta, out,
              tokrows_v, combrows_v, outbuf_v, ids_v, cix_v, tix_v, g_v, b_v,
              sem0, sem1, osem0, osem1, isem0, isem1):
    wid = lax.axis_index("s") * NC + lax.axis_index("c")
    base = wid * W
    sems = (sem0, sem1)
    osems = (osem0, osem1)
    isems = (isem0, isem1)

    pltpu.sync_copy(gamma, g_v)
    pltpu.sync_copy(beta, b_v)

    g_regs = [g_v[pl.ds(j * L, L)] for j in range(NJ)]
    b_regs = [b_v[pl.ds(j * L, L)] for j in range(NJ)]
    inv_d = jnp.float32(1.0 / D)

    def ids_start(c, b):
        pltpu.async_copy(ids3.at[:, pl.ds(base + c * CH, CH)], ids_v.at[b],
                         isems[b])

    def ids_wait(c, b):
        pltpu.make_async_copy(ids3.at[:, pl.ds(base + c * CH, CH)],
                              ids_v.at[b], isems[b]).wait()

    def gather_start(c, b):
        ids_wait(c, b)
        for j in range(CH // L):
            tix_v[b, pl.ds(j * L, L)] = ids_v[b, 0, pl.ds(j * L, L)]
            pvi = ids_v[b, 1, pl.ds(j * L, L)]
            tvi = ids_v[b, 2, pl.ds(j * L, L)]
            cix_v[b, pl.ds(j * L, L)] = pvi + pvi + tvi
        pltpu.async_copy(tok_tab.at[tix_v.at[b]], tokrows_v.at[b],
                         sems[b])
        pltpu.async_copy(comb_tab.at[cix_v.at[b]], combrows_v.at[b], sems[b])

        @pl.when(c + 2 < NCHUNK)
        def _():
            ids_start(c + 2, b)

    def wait_rows(b):
        pltpu.make_async_copy(
            tok_tab.at[tix_v.at[b]], tokrows_v.at[b], sems[b]).wait()
        pltpu.make_async_copy(
            comb_tab.at[cix_v.at[b]], combrows_v.at[b], sems[b]).wait()

    def wait_out(c, b):
        pltpu.make_async_copy(
            outbuf_v.at[b], out.at[pl.ds(base + c * CH, CH)], osems[b]).wait()

    perms = [jnp.reshape(lax.iota(jnp.int32, L) ^ jnp.int32(1 << p), (L, 1))
             for p in range(4)]
    _dnums = lax.GatherDimensionNumbers(
        offset_dims=(), collapsed_slice_dims=(0,), start_index_map=(0,))

    def _hsum(v):
        for p in perms:
            v = v + lax.gather(v, p, _dnums, slice_sizes=(1,),
                               mode=lax.GatherScatterMode.PROMISE_IN_BOUNDS)
        return v

    def ln_token(t, b):
        acc = [tokrows_v[b, t, pl.ds(j * L, L)] +
               combrows_v[b, t, pl.ds(j * L, L)] for j in range(NJ)]
        sv = _tree_add(acc)
        qv = _tree_add([a * a for a in acc])
        meanb = _hsum(sv) * inv_d
        varb = _hsum(qv) * inv_d - meanb * meanb
        yb = _rsqrt16(varb + jnp.float32(EPS))
        for j in range(NJ):
            u = (acc[j] - meanb) * yb
            outbuf_v[b, t, pl.ds(j * L, L)] = u * g_regs[j] + b_regs[j]

    def chunk(c, b):
        wait_rows(b)

        @pl.when(c >= 2)
        def _():
            wait_out(c - 2, b)

        @plsc.parallel_loop(0, CH, unroll=8)
        def _(t):
            ln_token(t, b)

        @pl.when(c + 2 < NCHUNK)
        def _():
            gather_start(c + 2, b)

        pltpu.async_copy(outbuf_v.at[b], out.at[pl.ds(base + c * CH, CH)],
                         osems[b])

    ids_start(0, 0)
    ids_start(1, 1)
    gather_start(0, 0)
    gather_start(1, 1)

    def outer(g, carry):
        chunk(g * 2, 0)
        chunk(g * 2 + 1, 1)
        return carry

    lax.fori_loop(0, NCHUNK // 2, outer, 0)
    wait_out(NCHUNK - 2, 0)
    wait_out(NCHUNK - 1, 1)


def kernel(token_ids, token_type_ids, token_pos, token_table, pos_table,
           type_table, gamma, beta):
    ids3 = jnp.stack([token_ids.reshape(-1).astype(jnp.int32),
                      token_pos.reshape(-1).astype(jnp.int32),
                      token_type_ids.reshape(-1).astype(jnp.int32)])
    comb = (pos_table[:, None, :] + type_table[None, :, :]).reshape(
        P * T, D)
    out = _sc_embed(ids3, token_table, comb, gamma, beta)
    return out.reshape(B, S, D)


# speedup vs baseline: 1.3988x; 1.3988x over previous
import functools

import jax
import jax.numpy as jnp
from jax import lax
from jax.experimental import pallas as pl
from jax.experimental.pallas import tpu as pltpu
from jax.experimental.pallas import tpu_sc as plsc

B, S, V, D, P, T = 1024, 200, 100000, 128, 512, 2
N = B * S
NC, NS, L = 2, 16, 16
NW = NC * NS
W = N // NW
CH = 128
NCHUNK = W // CH
NJ = D // L
EPS = 1e-12

_mesh = plsc.VectorSubcoreMesh(core_axis_name="c", subcore_axis_name="s")


def _rsqrt16(x):
    i = lax.bitcast_convert_type(x, jnp.int32)
    i = jnp.int32(0x5F3759DF) - lax.shift_right_logical(i, 1)
    y = lax.bitcast_convert_type(i, jnp.float32)
    h = x * jnp.float32(-0.5)
    for _ in range(2):
        y = y * (jnp.float32(1.5) + h * y * y)
    return y


def _tree_add(vs):
    vs = list(vs)
    while len(vs) > 1:
        vs = [vs[i] + vs[i + 1] for i in range(0, len(vs) - 1, 2)] + (
            [vs[-1]] if len(vs) % 2 else [])
    return vs[0]


@functools.partial(
    pl.kernel,
    out_type=jax.ShapeDtypeStruct((N, D), jnp.float32),
    mesh=_mesh,
    scratch_types=[
        pltpu.VMEM((2, CH, D), jnp.float32),
        pltpu.VMEM((2, CH, D), jnp.float32),
        pltpu.VMEM((2, CH, D), jnp.float32),
        pltpu.VMEM((2, 3, CH), jnp.int32),
        pltpu.VMEM((2, CH), jnp.int32),
        pltpu.VMEM((2, CH), jnp.int32),
        pltpu.VMEM((D,), jnp.float32),
        pltpu.VMEM((D,), jnp.float32),
        pltpu.SemaphoreType.DMA,
        pltpu.SemaphoreType.DMA,
        pltpu.SemaphoreType.DMA,
        pltpu.SemaphoreType.DMA,
        pltpu.SemaphoreType.DMA,
        pltpu.SemaphoreType.DMA,
    ],
)
def _sc_embed(ids3, tok_tab, comb_tab, gamma, beta, out,
              tokrows_v, combrows_v, outbuf_v, ids_v, cix_v, tix_v, g_v, b_v,
              sem0, sem1, osem0, osem1, isem0, isem1):
    wid = lax.axis_index("s") * NC + lax.axis_index("c")
    base = wid * W
    sems = (sem0, sem1)
    osems = (osem0, osem1)
    isems = (isem0, isem1)

    pltpu.sync_copy(gamma, g_v)
    pltpu.sync_copy(beta, b_v)

    g_regs = [g_v[pl.ds(j * L, L)] for j in range(NJ)]
    b_regs = [b_v[pl.ds(j * L, L)] for j in range(NJ)]
    inv_d = jnp.float32(1.0 / D)

    def ids_start(c, b):
        pltpu.async_copy(ids3.at[:, pl.ds(base + c * CH, CH)], ids_v.at[b],
                         isems[b])

    def ids_wait(c, b):
        pltpu.make_async_copy(ids3.at[:, pl.ds(base + c * CH, CH)],
                              ids_v.at[b], isems[b]).wait()

    def gather_start(c, b):
        ids_wait(c, b)
        for j in range(CH // L):
            tix_v[b, pl.ds(j * L, L)] = ids_v[b, 0, pl.ds(j * L, L)]
            pvi = ids_v[b, 1, pl.ds(j * L, L)]
            tvi = ids_v[b, 2, pl.ds(j * L, L)]
            cix_v[b, pl.ds(j * L, L)] = pvi + pvi + tvi
        pltpu.async_copy(tok_tab.at[tix_v.at[b]], tokrows_v.at[b],
                         sems[b])
        pltpu.async_copy(comb_tab.at[cix_v.at[b]], combrows_v.at[b], sems[b])

        @pl.when(c + 2 < NCHUNK)
        def _():
            ids_start(c + 2, b)

    def wait_rows(b):
        pltpu.make_async_copy(
            tok_tab.at[tix_v.at[b]], tokrows_v.at[b], sems[b]).wait()
        pltpu.make_async_copy(
            comb_tab.at[cix_v.at[b]], combrows_v.at[b], sems[b]).wait()

    def wait_out(c, b):
        pltpu.make_async_copy(
            outbuf_v.at[b], out.at[pl.ds(base + c * CH, CH)], osems[b]).wait()

    perms = [jnp.reshape(lax.iota(jnp.int32, L) ^ jnp.int32(1 << p), (L, 1))
             for p in range(4)]
    _dnums = lax.GatherDimensionNumbers(
        offset_dims=(), collapsed_slice_dims=(0,), start_index_map=(0,))

    def _hsum(v):
        for p in perms:
            v = v + lax.gather(v, p, _dnums, slice_sizes=(1,),
                               mode=lax.GatherScatterMode.PROMISE_IN_BOUNDS)
        return v

    def ln_token(t, b):
        acc = [tokrows_v[b, t, pl.ds(j * L, L)] +
               combrows_v[b, t, pl.ds(j * L, L)] for j in range(NJ)]
        sv = _tree_add(acc)
        qv = _tree_add([a * a for a in acc])
        meanb = _hsum(sv) * inv_d
        varb = _hsum(qv) * inv_d - meanb * meanb
        yb = _rsqrt16(varb + jnp.float32(EPS))
        for j in range(NJ):
            u = (acc[j] - meanb) * yb
            outbuf_v[b, t, pl.ds(j * L, L)] = u * g_regs[j] + b_regs[j]

    def chunk(c, b):
        wait_rows(b)

        @pl.when(c >= 2)
        def _():
            wait_out(c - 2, b)

        @plsc.parallel_loop(0, CH, unroll=4)
        def _(t):
            ln_token(t, b)

        @pl.when(c + 2 < NCHUNK)
        def _():
            gather_start(c + 2, b)

        pltpu.async_copy(outbuf_v.at[b], out.at[pl.ds(base + c * CH, CH)],
                         osems[b])

    ids_start(0, 0)
    ids_start(1, 1)
    gather_start(0, 0)
    gather_start(1, 1)

    def outer(g, carry):
        chunk(g * 2, 0)
        chunk(g * 2 + 1, 1)
        return carry

    lax.fori_loop(0, NCHUNK // 2, outer, 0)
    wait_out(NCHUNK - 2, 0)
    wait_out(NCHUNK - 1, 1)


def kernel(token_ids, token_type_ids, token_pos, token_table, pos_table,
           type_table, gamma, beta):
    ids3 = jnp.stack([token_ids.reshape(-1).astype(jnp.int32),
                      token_pos.reshape(-1).astype(jnp.int32),
                      token_type_ids.reshape(-1).astype(jnp.int32)])
    comb = (pos_table[:, None, :] + type_table[None, :, :]).reshape(
        P * T, D)
    out = _sc_embed(ids3, token_table, comb, gamma, beta)
    return out.reshape(B, S, D)
